# Initial kernel scaffold; baseline (speedup 1.0000x reference)
import jax
import jax.numpy as jnp
from jax.experimental import pallas as pl


def kernel(edge_index, senders_pos, receivers_pos, edge_dx_, edge_attr, vector_a, vector_b, vector_c, senders_v_t_, senders_v_tm1_, receivers_v_t_, receivers_v_tm1_, node_acc, node_latent, params):
    raise NotImplementedError("write your pallas kernel here")


# speedup vs baseline: 1.5955x; 1.5955x over previous
import functools

import jax
import jax.numpy as jnp
from jax import lax
from jax.experimental import pallas as pl
from jax.experimental.pallas import tpu as pltpu
from jax.experimental.pallas import tpu_sc as plsc

_E = 320000
_N = 10000
_D = 128

_NC = 2
_NS = 16
_NW = _NC * _NS
_CH = 128
_NCHUNKS = _E // _CH
_CPW = _NCHUNKS // _NW
_EXTRA = _NCHUNKS % _NW

def _sc_gather_body(s_hbm, r_hbm, nl_hbm, out_s, out_r,
                    nl_sh, sidx, ridx, srow, rrow, sem_s, sem_r):
    cid = lax.axis_index("c")
    sid = lax.axis_index("s")
    wid = sid * _NC + cid

    @pl.when(sid == 0)
    def _stage():
        pltpu.sync_copy(nl_hbm, nl_sh)

    plsc.subcore_barrier()

    nt = _CPW + jnp.where(wid < _EXTRA, 1, 0)

    def body(t, carry):
        off = (t * _NW + wid) * _CH
        pltpu.sync_copy(s_hbm.at[pl.ds(off, _CH)], sidx)
        pltpu.sync_copy(r_hbm.at[pl.ds(off, _CH)], ridx)
        cp_s = pltpu.async_copy(nl_sh.at[sidx], srow, sem_s)
        cp_r = pltpu.async_copy(nl_sh.at[ridx], rrow, sem_r)
        cp_s.wait()
        cp_r.wait()
        pltpu.sync_copy(srow, out_s.at[pl.ds(off, _CH)])
        pltpu.sync_copy(rrow, out_r.at[pl.ds(off, _CH)])
        return carry

    lax.fori_loop(0, nt, body, 0)


def _sc_scatter_body(r_hbm, fx_hbm, fy_hbm, fz_hbm, zeros_hbm, out,
                     acc_sh, ridx, fidx, fval):
    cid = lax.axis_index("c")
    sid = lax.axis_index("s")
    wid = sid * _NC + cid

    @pl.when(sid == 0)
    def _init():
        pltpu.sync_copy(zeros_hbm, acc_sh)

    plsc.subcore_barrier()

    nt = _CPW + jnp.where(wid < _EXTRA, 1, 0)

    def body(t, carry):
        off = (t * _NW + wid) * _CH
        pltpu.sync_copy(r_hbm.at[pl.ds(off, _CH)], ridx)
        for c, f_hbm in enumerate((fx_hbm, fy_hbm, fz_hbm)):
            def idx_body(g, u, _c=c):
                rv = ridx[pl.ds(g * 16, 16)]
                fidx[pl.ds(g * 16, 16)] = rv * 3 + _c
                return u
            lax.fori_loop(0, _CH // 16, idx_body, 0)
            pltpu.sync_copy(f_hbm.at[pl.ds(off, _CH)], fval)
            pltpu.sync_copy(fval, acc_sh.at[fidx], add=True)
        return carry

    lax.fori_loop(0, nt, body, 0)
    plsc.subcore_barrier()

    @pl.when(sid == 0)
    def _flush():
        pltpu.sync_copy(acc_sh, out.at[cid])


@functools.cache
def _sc_kernels():
    mesh = plsc.VectorSubcoreMesh(core_axis_name="c", subcore_axis_name="s",
                                  num_cores=_NC, num_subcores=_NS)
    gather = pl.kernel(
        _sc_gather_body,
        out_type=(
            jax.ShapeDtypeStruct((_E, _D), jnp.float32),
            jax.ShapeDtypeStruct((_E, _D), jnp.float32),
        ),
        mesh=mesh,
        scratch_types=[
            pltpu.VMEM_SHARED((_N, _D), jnp.float32),
            pltpu.VMEM((_CH,), jnp.int32),
            pltpu.VMEM((_CH,), jnp.int32),
            pltpu.VMEM((_CH, _D), jnp.float32),
            pltpu.VMEM((_CH, _D), jnp.float32),
            pltpu.SemaphoreType.DMA,
            pltpu.SemaphoreType.DMA,
        ],
    )
    scatter = pl.kernel(
        _sc_scatter_body,
        out_type=jax.ShapeDtypeStruct((_NC, _N * 3), jnp.float32),
        mesh=mesh,
        scratch_types=[
            pltpu.VMEM_SHARED((_N * 3,), jnp.float32),
            pltpu.VMEM((_CH,), jnp.int32),
            pltpu.VMEM((_CH,), jnp.int32),
            pltpu.VMEM((_CH,), jnp.float32),
        ],
    )
    return gather, scatter


_BE = 1600


def _edge_body(va_r, vb_r, vc_r, dx_r, attr_r, svt_r, svtm1_r, rvt_r, rvtm1_r,
               sr_r, rr_r,
               wf1, bf1, wf2, bf2, gf, bnf,
               we1, be1, we2, be2, ge, bne,
               wia, wib, wic, bi1, wi2, bi2, gi, bni,
               wc1, bc1, wc2, bc2,
               out_il, out_fij):
    f32 = jnp.float32
    va = va_r[...]
    vb = vb_r[...]
    vc = vc_r[...]

    def proj(v):
        return (jnp.sum(va * v, axis=1, keepdims=True),
                jnp.sum(vb * v, axis=1, keepdims=True),
                jnp.sum(vc * v, axis=1, keepdims=True))

    w1 = wf1[...]

    def feat_pre(vt, vtm1, sign):
        p0, p1, p2 = proj(vt)
        q0, q1, q2 = proj(vtm1)
        acc = (p0 * w1[0:1] + p1 * w1[1:2] + p2 * w1[2:3]
               + q0 * w1[3:4] + q1 * w1[4:5] + q2 * w1[5:6])
        return sign * acc + bf1[...]

    def ln(y, g, bn):
        mu = jnp.mean(y, axis=1, keepdims=True)
        yc = y - mu
        var = jnp.mean(yc * yc, axis=1, keepdims=True)
        return yc * lax.rsqrt(var + 1e-5) * g[...] + bn[...]

    def mm(x, w):
        return jnp.dot(x, w[...], preferred_element_type=f32)

    hs = jnp.maximum(feat_pre(svt_r[...], svtm1_r[...], 1.0), 0.0)
    hr = jnp.maximum(feat_pre(rvt_r[...], rvtm1_r[...], -1.0), 0.0)
    sl = ln(mm(hs, wf2) + bf2[...], gf, bnf)
    rl = ln(mm(hr, wf2) + bf2[...], gf, bnf)

    dx = dx_r[...]
    nrm = jnp.sqrt(jnp.sum(dx * dx, axis=1, keepdims=True))
    we = we1[...]
    he = jnp.maximum(nrm * we[0:1] + attr_r[...] * we[1:2] + be1[...], 0.0)
    el = ln(mm(he, we2) + be2[...], ge, bne)

    nsum = sr_r[...] + rr_r[...]
    h = jnp.maximum(mm(sl + rl, wia) + mm(nsum, wib) + mm(el, wic) + bi1[...],
                    0.0)
    il = ln(mm(h, wi2) + bi2[...], gi, bni)
    out_il[...] = il

    h2 = jnp.maximum(mm(il, wc1) + bc1[...], 0.0)
    coeff = mm(h2, wc2) + bc2[...]
    out_fij[...] = coeff[:, 0:1] * va + coeff[:, 1:2] * vb + coeff[:, 2:3] * vc


def _edge_call(per_edge, weights):
    grid = (_E // _BE,)

    def eb(cols):
        return pl.BlockSpec((_BE, cols), lambda i: (i, 0))

    def wb(shape):
        return pl.BlockSpec(shape, lambda i: (0, 0))

    in_specs = ([eb(3)] * 4 + [eb(1)] + [eb(3)] * 4 + [eb(_D)] * 2
                + [wb(w.shape) for w in weights])
    return pl.pallas_call(
        _edge_body,
        grid=grid,
        in_specs=in_specs,
        out_specs=[eb(_D), eb(3)],
        out_shape=[
            jax.ShapeDtypeStruct((_E, _D), jnp.float32),
            jax.ShapeDtypeStruct((_E, 3), jnp.float32),
        ],
    )(*per_edge, *weights)


def _node_body(nl_r, nacc_r, p0_r, p1_r,
               wm1, bm1, wm2, bm2, wx1, bx1, wx2, bx2, out):
    f32 = jnp.float32
    nl = nl_r[...]
    hm = jnp.maximum(jnp.dot(nl, wm1[...], preferred_element_type=f32)
                     + bm1[...], 0.0)
    m = jnp.dot(hm, wm2[...], preferred_element_type=f32) + bm2[...]
    hx = jnp.maximum(jnp.dot(nl, wx1[...], preferred_element_type=f32)
                     + bx1[...], 0.0)
    fx = jnp.dot(hx, wx2[...], preferred_element_type=f32) + bx2[...]
    out[...] = m * nacc_r[...] + p0_r[...] + p1_r[...] - fx


def _node_call(node_latent, node_acc, p0, p1, weights):
    return pl.pallas_call(
        _node_body,
        out_shape=jax.ShapeDtypeStruct((_N, 3), jnp.float32),
    )(node_latent, node_acc, p0, p1, *weights)


def kernel(edge_index, senders_pos, receivers_pos, edge_dx_, edge_attr,
           vector_a, vector_b, vector_c, senders_v_t_, senders_v_tm1_,
           receivers_v_t_, receivers_v_tm1_, node_acc, node_latent, params):
    senders = edge_index[0]
    receivers = edge_index[1]

    sc_gather, sc_scatter = _sc_kernels()
    s_rows, r_rows = sc_gather(senders, receivers, node_latent)

    pf = params["edge_feat"]
    pe = params["edge"]
    pi = params["inter"]
    pc = params["i1"]
    wi1 = pi["W1"]
    edge_weights = [
        pf["W1"], pf["b1"][None, :], pf["W2"], pf["b2"][None, :],
        pf["g"][None, :], pf["bn"][None, :],
        pe["W1"], pe["b1"][None, :], pe["W2"], pe["b2"][None, :],
        pe["g"][None, :], pe["bn"][None, :],
        wi1[0:_D], wi1[_D:2 * _D], wi1[2 * _D:3 * _D], pi["b1"][None, :],
        pi["W2"], pi["b2"][None, :], pi["g"][None, :], pi["bn"][None, :],
        pc["W1"], pc["b1"][None, :], pc["W2"], pc["b2"][None, :],
    ]
    per_edge = [vector_a, vector_b, vector_c, edge_dx_, edge_attr,
                senders_v_t_, senders_v_tm1_, receivers_v_t_, receivers_v_tm1_,
                s_rows, r_rows]
    il, fij = _edge_call(per_edge, edge_weights)

    zeros = jnp.zeros((_N * 3,), jnp.float32)
    partials = sc_scatter(receivers, fij[:, 0], fij[:, 1], fij[:, 2], zeros)
    p = partials.reshape(_NC, _N, 3)

    pm = params["m"]
    px = params["fext"]
    node_weights = [
        pm["W1"], pm["b1"][None, :], pm["W2"], pm["b2"][None, :],
        px["W1"], px["b1"][None, :], px["W2"], px["b2"][None, :],
    ]
    residual = _node_call(node_latent, node_acc, p[0], p[1], node_weights)
    return (residual, il)


# speedup vs baseline: 3.2536x; 2.0392x over previous
import functools

import jax
import jax.numpy as jnp
from jax import lax
from jax.experimental import pallas as pl
from jax.experimental.pallas import tpu as pltpu
from jax.experimental.pallas import tpu_sc as plsc

_E = 320000
_N = 10000
_D = 128

_NC = 2
_NS = 16
_NW = _NC * _NS
_CH = 128
_NCHUNKS = _E // _CH
_CPW = _NCHUNKS // _NW
_EXTRA = _NCHUNKS % _NW

def _sc_gather_body(s_hbm, r_hbm, nl_hbm, out_s, out_r,
                    nl_sh, sidx, ridx, srow, rrow, sem_s, sem_r):
    cid = lax.axis_index("c")
    sid = lax.axis_index("s")
    wid = sid * _NC + cid

    @pl.when(sid == 0)
    def _stage():
        pltpu.sync_copy(nl_hbm, nl_sh)

    plsc.subcore_barrier()

    nt = _CPW + jnp.where(wid < _EXTRA, 1, 0)

    def body(t, carry):
        off = (t * _NW + wid) * _CH
        pltpu.sync_copy(s_hbm.at[pl.ds(off, _CH)], sidx)
        pltpu.sync_copy(r_hbm.at[pl.ds(off, _CH)], ridx)
        cp_s = pltpu.async_copy(nl_sh.at[sidx], srow, sem_s)
        cp_r = pltpu.async_copy(nl_sh.at[ridx], rrow, sem_r)
        cp_s.wait()
        cp_r.wait()
        pltpu.sync_copy(srow, out_s.at[pl.ds(off, _CH)])
        pltpu.sync_copy(rrow, out_r.at[pl.ds(off, _CH)])
        return carry

    lax.fori_loop(0, nt, body, 0)


def _sc_scatter_body(r_hbm, fx_hbm, fy_hbm, fz_hbm, zeros_hbm, out,
                     acc_sh, ridx, fidx, fval):
    cid = lax.axis_index("c")
    sid = lax.axis_index("s")
    wid = sid * _NC + cid

    @pl.when(sid == 0)
    def _init():
        pltpu.sync_copy(zeros_hbm, acc_sh)

    plsc.subcore_barrier()

    nt = _CPW + jnp.where(wid < _EXTRA, 1, 0)

    def body(t, carry):
        off = (t * _NW + wid) * _CH
        pltpu.sync_copy(r_hbm.at[pl.ds(off, _CH)], ridx)
        for c, f_hbm in enumerate((fx_hbm, fy_hbm, fz_hbm)):
            def idx_body(g, u, _c=c):
                rv = ridx[pl.ds(g * 16, 16)]
                fidx[pl.ds(g * 16, 16)] = rv * 3 + _c
                return u
            lax.fori_loop(0, _CH // 16, idx_body, 0)
            pltpu.sync_copy(f_hbm.at[pl.ds(off, _CH)], fval)
            pltpu.sync_copy(fval, acc_sh.at[fidx], add=True)
        return carry

    lax.fori_loop(0, nt, body, 0)
    plsc.subcore_barrier()

    @pl.when(sid == 0)
    def _flush():
        pltpu.sync_copy(acc_sh, out.at[cid])


@functools.cache
def _sc_kernels():
    mesh = plsc.VectorSubcoreMesh(core_axis_name="c", subcore_axis_name="s",
                                  num_cores=_NC, num_subcores=_NS)
    gather = pl.kernel(
        _sc_gather_body,
        out_type=(
            jax.ShapeDtypeStruct((_E, _D), jnp.float32),
            jax.ShapeDtypeStruct((_E, _D), jnp.float32),
        ),
        mesh=mesh,
        scratch_types=[
            pltpu.VMEM_SHARED((_N, _D), jnp.float32),
            pltpu.VMEM((_CH,), jnp.int32),
            pltpu.VMEM((_CH,), jnp.int32),
            pltpu.VMEM((_CH, _D), jnp.float32),
            pltpu.VMEM((_CH, _D), jnp.float32),
            pltpu.SemaphoreType.DMA,
            pltpu.SemaphoreType.DMA,
        ],
    )
    scatter = pl.kernel(
        _sc_scatter_body,
        out_type=jax.ShapeDtypeStruct((_NC, _N * 3), jnp.float32),
        mesh=mesh,
        scratch_types=[
            pltpu.VMEM_SHARED((_N * 3,), jnp.float32),
            pltpu.VMEM((_CH,), jnp.int32),
            pltpu.VMEM((_CH,), jnp.int32),
            pltpu.VMEM((_CH,), jnp.float32),
        ],
    )
    return gather, scatter


_BE = 1600


def _edge_body(va_r, vb_r, vc_r, dx_r, attr_r, svt_r, svtm1_r, rvt_r, rvtm1_r,
               sr_r, rr_r,
               wf1, bf1, wf2, bf2, gf, bnf,
               we1, be1, we2, be2, ge, bne,
               wia, wib, wic, bi1, wi2, bi2, gi, bni,
               wc1, bc1, wc2t, bc2t,
               out_il, out_fijt):
    f32 = jnp.float32
    va = va_r[0]
    vb = vb_r[0]
    vc = vc_r[0]

    def proj6(vt, vtm1):
        return jnp.concatenate([
            jnp.sum(va * vt, axis=0, keepdims=True),
            jnp.sum(vb * vt, axis=0, keepdims=True),
            jnp.sum(vc * vt, axis=0, keepdims=True),
            jnp.sum(va * vtm1, axis=0, keepdims=True),
            jnp.sum(vb * vtm1, axis=0, keepdims=True),
            jnp.sum(vc * vtm1, axis=0, keepdims=True)], axis=0)

    dn_t = (((0,), (0,)), ((), ()))

    def ln(y, g, bn):
        mu = jnp.mean(y, axis=1, keepdims=True)
        yc = y - mu
        var = jnp.mean(yc * yc, axis=1, keepdims=True)
        return yc * lax.rsqrt(var + 1e-5) * g[...] + bn[...]

    def mm(x, w):
        return jnp.dot(x, w[...], preferred_element_type=f32)

    sft = proj6(svt_r[0], svtm1_r[0])
    rft = proj6(rvt_r[0], rvtm1_r[0])
    hs = jnp.maximum(
        lax.dot_general(sft, wf1[...], dn_t, preferred_element_type=f32)
        + bf1[...], 0.0)
    hr = jnp.maximum(
        -lax.dot_general(rft, wf1[...], dn_t, preferred_element_type=f32)
        + bf1[...], 0.0)
    sl = ln(mm(hs, wf2) + bf2[...], gf, bnf)
    rl = ln(mm(hr, wf2) + bf2[...], gf, bnf)

    dx = dx_r[0]
    nrm = jnp.sqrt(jnp.sum(dx * dx, axis=0, keepdims=True))
    ef = jnp.concatenate([nrm, attr_r[0]], axis=0)
    he = jnp.maximum(
        lax.dot_general(ef, we1[...], dn_t, preferred_element_type=f32)
        + be1[...], 0.0)
    el = ln(mm(he, we2) + be2[...], ge, bne)

    nsum = sr_r[...] + rr_r[...]
    h = jnp.maximum(mm(sl + rl, wia) + mm(nsum, wib) + mm(el, wic) + bi1[...],
                    0.0)
    il = ln(mm(h, wi2) + bi2[...], gi, bni)
    out_il[...] = il

    h2 = jnp.maximum(mm(il, wc1) + bc1[...], 0.0)
    coeft = lax.dot_general(wc2t[...], h2, (((1,), (1,)), ((), ())),
                            preferred_element_type=f32) + bc2t[...]
    out_fijt[0] = coeft[0:1] * va + coeft[1:2] * vb + coeft[2:3] * vc


def _edge_call(per_edge, weights):
    nb = _E // _BE
    grid = (nb,)

    def eb(rows):
        return pl.BlockSpec((1, rows, _BE), lambda i: (i, 0, 0))

    def rb(cols):
        return pl.BlockSpec((_BE, cols), lambda i: (i, 0))

    def wb(shape):
        return pl.BlockSpec(shape, lambda i: (0,) * len(shape))

    in_specs = ([eb(3)] * 4 + [eb(1)] + [eb(3)] * 4 + [rb(_D)] * 2
                + [wb(w.shape) for w in weights])
    return pl.pallas_call(
        _edge_body,
        grid=grid,
        in_specs=in_specs,
        out_specs=[rb(_D), eb(3)],
        out_shape=[
            jax.ShapeDtypeStruct((_E, _D), jnp.float32),
            jax.ShapeDtypeStruct((nb, 3, _BE), jnp.float32),
        ],
    )(*per_edge, *weights)


def _node_body(nl_r, nacc_r, p0_r, p1_r,
               wm1, bm1, wm2, bm2, wx1, bx1, wx2, bx2, out):
    f32 = jnp.float32
    nl = nl_r[...]
    hm = jnp.maximum(jnp.dot(nl, wm1[...], preferred_element_type=f32)
                     + bm1[...], 0.0)
    m = jnp.dot(hm, wm2[...], preferred_element_type=f32) + bm2[...]
    hx = jnp.maximum(jnp.dot(nl, wx1[...], preferred_element_type=f32)
                     + bx1[...], 0.0)
    fx = jnp.dot(hx, wx2[...], preferred_element_type=f32) + bx2[...]
    out[...] = m * nacc_r[...] + p0_r[...] + p1_r[...] - fx


def _node_call(node_latent, node_acc, p0, p1, weights):
    return pl.pallas_call(
        _node_body,
        out_shape=jax.ShapeDtypeStruct((_N, 3), jnp.float32),
    )(node_latent, node_acc, p0, p1, *weights)


def kernel(edge_index, senders_pos, receivers_pos, edge_dx_, edge_attr,
           vector_a, vector_b, vector_c, senders_v_t_, senders_v_tm1_,
           receivers_v_t_, receivers_v_tm1_, node_acc, node_latent, params):
    senders = edge_index[0]
    receivers = edge_index[1]

    sc_gather, sc_scatter = _sc_kernels()
    s_rows, r_rows = sc_gather(senders, receivers, node_latent)

    pf = params["edge_feat"]
    pe = params["edge"]
    pi = params["inter"]
    pc = params["i1"]
    wi1 = pi["W1"]
    edge_weights = [
        pf["W1"], pf["b1"][None, :], pf["W2"], pf["b2"][None, :],
        pf["g"][None, :], pf["bn"][None, :],
        pe["W1"], pe["b1"][None, :], pe["W2"], pe["b2"][None, :],
        pe["g"][None, :], pe["bn"][None, :],
        wi1[0:_D], wi1[_D:2 * _D], wi1[2 * _D:3 * _D], pi["b1"][None, :],
        pi["W2"], pi["b2"][None, :], pi["g"][None, :], pi["bn"][None, :],
        pc["W1"], pc["b1"][None, :], pc["W2"].T, pc["b2"][:, None],
    ]
    def t3(x):
        return jnp.transpose(x.reshape(_E // _BE, _BE, x.shape[1]), (0, 2, 1))

    per_edge = [t3(vector_a), t3(vector_b), t3(vector_c), t3(edge_dx_),
                t3(edge_attr), t3(senders_v_t_), t3(senders_v_tm1_),
                t3(receivers_v_t_), t3(receivers_v_tm1_),
                s_rows, r_rows]
    il, fijt3 = _edge_call(per_edge, edge_weights)

    zeros = jnp.zeros((_N * 3,), jnp.float32)
    partials = sc_scatter(receivers,
                          fijt3[:, 0, :].reshape(_E),
                          fijt3[:, 1, :].reshape(_E),
                          fijt3[:, 2, :].reshape(_E), zeros)
    p = partials.reshape(_NC, _N, 3)

    pm = params["m"]
    px = params["fext"]
    node_weights = [
        pm["W1"], pm["b1"][None, :], pm["W2"], pm["b2"][None, :],
        px["W1"], px["b1"][None, :], px["W2"], px["b2"][None, :],
    ]
    residual = _node_call(node_latent, node_acc, p[0], p[1], node_weights)
    return (residual, il)
